# baseline (speedup 1.0000x reference)
import functools

import jax
import jax.numpy as jnp
from jax import lax
from jax.experimental import pallas as pl
from jax.experimental.pallas import tpu as pltpu
from jax.experimental.pallas import tpu_sc as plsc

_EMB = 64
_BATCH = 4096
_HIST = 50
_NTOT = _BATCH * _HIST
_VOCAB = 1000000

_info = plsc.get_sparse_core_info()
_NC, _NS = _info.num_cores, _info.num_subcores
_NW = _NC * _NS
_B_PER_W = _NTOT // _NW
_CHUNK = 800
_NCHUNK = _B_PER_W // _CHUNK

_NBLK_FULL = _VOCAB // 128
_BPW = 244
_BLK_F32 = 128 * _EMB

_mesh = plsc.VectorSubcoreMesh(core_axis_name="c", subcore_axis_name="s")


@functools.partial(
    pl.kernel,
    mesh=_mesh,
    out_type=jax.ShapeDtypeStruct((_NTOT,), jnp.int32),
    scratch_types=[
        pltpu.VMEM((8, 128), jnp.int32),
    ],
)
def _detile_idx(idxt_hbm, out_hbm, tile_v):
    wid = lax.axis_index("s") * _NC + lax.axis_index("c")
    col = wid * 128
    for a in range(7):
        rows = 8 if a < 6 else 2
        pltpu.sync_copy(
            idxt_hbm.at[pl.ds(a * 8, rows), pl.ds(col, 128)],
            tile_v.at[pl.ds(0, rows)],
        )
        for s in range(rows):
            pltpu.sync_copy(
                tile_v.at[s],
                out_hbm.at[pl.ds((a * 8 + s) * _BATCH + col, 128)],
            )


@functools.partial(
    pl.kernel,
    mesh=_mesh,
    out_type=jax.ShapeDtypeStruct((_VOCAB * _EMB,), jnp.float32),
    scratch_types=[
        pltpu.VMEM((2, _EMB, 129), jnp.float32),
        pltpu.VMEM((2, _BLK_F32), jnp.float32),
        pltpu.SemaphoreType.DMA,
        pltpu.SemaphoreType.DMA,
        pltpu.SemaphoreType.DMA,
        pltpu.SemaphoreType.DMA,
    ],
    compiler_params=pltpu.CompilerParams(needs_layout_passes=False),
)
def _linearize_table(wt_hbm, wtail_hbm, out_hbm, bin_v, bout_v, isem0, isem1,
                     osem0, osem1):
    wid = lax.axis_index("s") * _NC + lax.axis_index("c")
    blk0 = wid * _BPW
    iota16 = lax.iota(jnp.int32, 16)
    isems = (isem0, isem1)
    osems = (osem0, osem1)

    def start_in(j, b):
        pltpu.async_copy(
            wt_hbm.at[:, pl.ds(pl.multiple_of(j * 128, 128), 128)],
            bin_v.at[b, :, pl.ds(0, 128)],
            isems[b],
        )

    def wait_in(b):
        pltpu.make_async_copy(
            wt_hbm.at[:, pl.ds(0, 128)],
            bin_v.at[b, :, pl.ds(0, 128)],
            isems[b],
        ).wait()

    def start_out(j, b):
        pltpu.async_copy(
            bout_v.at[b],
            out_hbm.at[pl.ds(pl.multiple_of(j * _BLK_F32, 8), _BLK_F32)],
            osems[b],
        )

    def wait_out(b):
        pltpu.make_async_copy(
            bout_v.at[b], out_hbm.at[pl.ds(0, _BLK_F32)], osems[b]
        ).wait()

    row_ids = [iota16 + (16 * s) for s in range(4)]

    def transpose(b):
        def tcol(i, _):
            c0 = i * 16
            cbase = jnp.full((16,), c0, jnp.int32)
            for dc in range(16):
                cvec = cbase + dc
                for s in range(4):
                    vals = plsc.load_gather(bin_v.at[b], [row_ids[s], cvec])
                    bout_v[b, pl.ds(c0 * 64 + dc * 64 + 16 * s, 16)] = vals
            return _

        lax.fori_loop(0, 8, tcol, 0)

    start_in(blk0, 0)
    start_in(blk0 + 1, 1)

    def iter_body(i, b, first):
        wait_in(b)
        if not first:
            wait_out(b)
        transpose(b)
        start_out(blk0 + i, b)

        @pl.when(i + 2 < _BPW)
        def _():
            start_in(blk0 + i + 2, b)

    iter_body(0, 0, True)
    iter_body(1, 1, True)

    def body(k, _):
        iter_body(2 * k, 0, False)
        iter_body(2 * k + 1, 1, False)
        return _

    lax.fori_loop(1, _BPW // 2, body, 0)
    wait_out(0)
    wait_out(1)

    for w in range(4):

        @pl.when(wid == w)
        def _():
            jt = _NBLK_FULL - 4 + w
            start_in(jt, 0)
            wait_in(0)
            transpose(0)
            start_out(jt, 0)
            wait_out(0)

    @pl.when(wid == 4)
    def _():
        pltpu.sync_copy(wtail_hbm, bin_v.at[0, :, pl.ds(0, 128)])

        def tq(q, col):
            for p in range(2):
                cvec = col + p
                for s in range(4):
                    vals = plsc.load_gather(
                        bin_v.at[0], [iota16 + (16 * s), cvec]
                    )
                    bout_v[0, pl.ds(q * 128 + p * 64 + 16 * s, 16)] = vals
            return col + 2

        lax.fori_loop(0, 32, tq, jnp.full((16,), 64, jnp.int32))
        pltpu.sync_copy(
            bout_v.at[0, pl.ds(0, 32 * 128)],
            out_hbm.at[pl.ds(_VOCAB * _EMB - 32 * 128, 32 * 128)],
        )


@functools.partial(
    pl.kernel,
    mesh=_mesh,
    out_type=jax.ShapeDtypeStruct((_NTOT, _EMB), jnp.float32),
    scratch_types=[
        pltpu.VMEM((_B_PER_W,), jnp.int32),
        pltpu.VMEM((2, _CHUNK, _EMB), jnp.float32),
        pltpu.SemaphoreType.DMA,
        pltpu.SemaphoreType.DMA,
    ],
    compiler_params=pltpu.CompilerParams(use_tc_tiling_on_sc=False),
)
def _emb_lookup(idx_hbm, table_hbm, out_hbm, idx_v, rows_v, gsem0, gsem1):
    wid = lax.axis_index("s") * _NC + lax.axis_index("c")
    base = wid * _B_PER_W
    pltpu.sync_copy(idx_hbm.at[pl.ds(base, _B_PER_W)], idx_v)

    pltpu.async_copy(
        table_hbm.at[idx_v.at[pl.ds(0, _CHUNK)]], rows_v.at[0], gsem0
    )

    def body(p, _):
        c0 = 2 * p
        pltpu.async_copy(
            table_hbm.at[idx_v.at[pl.ds((c0 + 1) * _CHUNK, _CHUNK)]],
            rows_v.at[1],
            gsem1,
        )
        pltpu.make_async_copy(
            table_hbm.at[idx_v.at[pl.ds(0, _CHUNK)]], rows_v.at[0], gsem0
        ).wait()
        pltpu.sync_copy(
            rows_v.at[0], out_hbm.at[pl.ds(base + c0 * _CHUNK, _CHUNK)]
        )
        nxt = lax.min(c0 + 2, _NCHUNK - 2)
        pltpu.async_copy(
            table_hbm.at[idx_v.at[pl.ds(nxt * _CHUNK, _CHUNK)]],
            rows_v.at[0],
            gsem0,
        )
        pltpu.make_async_copy(
            table_hbm.at[idx_v.at[pl.ds(0, _CHUNK)]], rows_v.at[1], gsem1
        ).wait()
        pltpu.sync_copy(
            rows_v.at[1], out_hbm.at[pl.ds(base + (c0 + 1) * _CHUNK, _CHUNK)]
        )
        return _

    lax.fori_loop(0, _NCHUNK // 2, body, 0)
    pltpu.make_async_copy(
        table_hbm.at[idx_v.at[pl.ds(0, _CHUNK)]], rows_v.at[0], gsem0
    ).wait()


def kernel(input, weight):
    idxt = jnp.transpose(input.astype(jnp.int32))
    flat = _detile_idx(idxt)
    wt = jnp.transpose(weight)
    wtail = lax.slice(wt, (0, _VOCAB - 128), (_EMB, _VOCAB))
    lin = _linearize_table(wt, wtail)
    table = jnp.reshape(lin, (_VOCAB, _EMB))
    out = _emb_lookup(flat, table)
    out3 = jnp.reshape(out, (_HIST, _BATCH, _EMB))
    return jnp.transpose(out3, (1, 0, 2))


# speedup vs baseline: 4.3791x; 4.3791x over previous
import functools

import jax
import jax.numpy as jnp
from jax import lax
from jax.experimental import pallas as pl
from jax.experimental.pallas import tpu as pltpu
from jax.experimental.pallas import tpu_sc as plsc

_EMB = 64
_BATCH = 4096
_HIST = 50
_NTOT = _BATCH * _HIST
_VOCAB = 1000000

_info = plsc.get_sparse_core_info()
_NC, _NS = _info.num_cores, _info.num_subcores
_NW = _NC * _NS
_B_PER_W = _NTOT // _NW
_CHUNK = 800
_NCHUNK = _B_PER_W // _CHUNK

_NBLK_FULL = _VOCAB // 128
_BPW = 244
_BLK_F32 = 128 * _EMB

_mesh = plsc.VectorSubcoreMesh(core_axis_name="c", subcore_axis_name="s")


@functools.partial(
    pl.kernel,
    mesh=_mesh,
    out_type=jax.ShapeDtypeStruct((_NTOT,), jnp.int32),
    scratch_types=[
        pltpu.VMEM((8, 128), jnp.int32),
    ],
)
def _detile_idx(idxt_hbm, out_hbm, tile_v):
    wid = lax.axis_index("s") * _NC + lax.axis_index("c")
    col = wid * 128
    for a in range(7):
        rows = 8 if a < 6 else 2
        pltpu.sync_copy(
            idxt_hbm.at[pl.ds(a * 8, rows), pl.ds(col, 128)],
            tile_v.at[pl.ds(0, rows)],
        )
        for s in range(rows):
            pltpu.sync_copy(
                tile_v.at[s],
                out_hbm.at[pl.ds((a * 8 + s) * _BATCH + col, 128)],
            )


@functools.partial(
    pl.kernel,
    mesh=_mesh,
    out_type=jax.ShapeDtypeStruct((_VOCAB * _EMB,), jnp.float32),
    scratch_types=[
        pltpu.VMEM((2, _EMB, 129), jnp.float32),
        pltpu.VMEM((2, _BLK_F32), jnp.float32),
        pltpu.SemaphoreType.DMA,
        pltpu.SemaphoreType.DMA,
        pltpu.SemaphoreType.DMA,
        pltpu.SemaphoreType.DMA,
    ],
    compiler_params=pltpu.CompilerParams(needs_layout_passes=False),
)
def _linearize_table(wt_hbm, wtail_hbm, out_hbm, bin_v, bout_v, isem0, isem1,
                     osem0, osem1):
    wid = lax.axis_index("s") * _NC + lax.axis_index("c")
    blk0 = wid * _BPW
    iota16 = lax.iota(jnp.int32, 16)
    isems = (isem0, isem1)
    osems = (osem0, osem1)

    def start_in(j, b):
        pltpu.async_copy(
            wt_hbm.at[:, pl.ds(pl.multiple_of(j * 128, 128), 128)],
            bin_v.at[b, :, pl.ds(0, 128)],
            isems[b],
        )

    def wait_in(b):
        pltpu.make_async_copy(
            wt_hbm.at[:, pl.ds(0, 128)],
            bin_v.at[b, :, pl.ds(0, 128)],
            isems[b],
        ).wait()

    def start_out(j, b):
        pltpu.async_copy(
            bout_v.at[b],
            out_hbm.at[pl.ds(pl.multiple_of(j * _BLK_F32, 8), _BLK_F32)],
            osems[b],
        )

    def wait_out(b):
        pltpu.make_async_copy(
            bout_v.at[b], out_hbm.at[pl.ds(0, _BLK_F32)], osems[b]
        ).wait()

    row_ids = [iota16 + (16 * s) for s in range(4)]

    def transpose(b):
        @functools.partial(plsc.parallel_loop, 0, 128, unroll=8)
        def _(col):
            cvec = jnp.full((16,), col, jnp.int32)
            for s in range(4):
                vals = plsc.load_gather(bin_v.at[b], [row_ids[s], cvec])
                bout_v[b, pl.ds(col * 64 + 16 * s, 16)] = vals

    start_in(blk0, 0)
    start_in(blk0 + 1, 1)

    def iter_body(i, b, first):
        wait_in(b)
        if not first:
            wait_out(b)
        transpose(b)
        start_out(blk0 + i, b)

        @pl.when(i + 2 < _BPW)
        def _():
            start_in(blk0 + i + 2, b)

    iter_body(0, 0, True)
    iter_body(1, 1, True)

    def body(k, _):
        iter_body(2 * k, 0, False)
        iter_body(2 * k + 1, 1, False)
        return _

    lax.fori_loop(1, _BPW // 2, body, 0)
    wait_out(0)
    wait_out(1)

    for w in range(4):

        @pl.when(wid == w)
        def _():
            jt = _NBLK_FULL - 4 + w
            start_in(jt, 0)
            wait_in(0)
            transpose(0)
            start_out(jt, 0)
            wait_out(0)

    @pl.when(wid == 4)
    def _():
        pltpu.sync_copy(wtail_hbm, bin_v.at[0, :, pl.ds(0, 128)])

        def tq(q, col):
            for p in range(2):
                cvec = col + p
                for s in range(4):
                    vals = plsc.load_gather(
                        bin_v.at[0], [iota16 + (16 * s), cvec]
                    )
                    bout_v[0, pl.ds(q * 128 + p * 64 + 16 * s, 16)] = vals
            return col + 2

        lax.fori_loop(0, 32, tq, jnp.full((16,), 64, jnp.int32))
        pltpu.sync_copy(
            bout_v.at[0, pl.ds(0, 32 * 128)],
            out_hbm.at[pl.ds(_VOCAB * _EMB - 32 * 128, 32 * 128)],
        )


@functools.partial(
    pl.kernel,
    mesh=_mesh,
    out_type=jax.ShapeDtypeStruct((_NTOT, _EMB), jnp.float32),
    scratch_types=[
        pltpu.VMEM((_B_PER_W,), jnp.int32),
        pltpu.VMEM((2, _CHUNK, _EMB), jnp.float32),
        pltpu.SemaphoreType.DMA,
        pltpu.SemaphoreType.DMA,
    ],
    compiler_params=pltpu.CompilerParams(use_tc_tiling_on_sc=False),
)
def _emb_lookup(idx_hbm, table_hbm, out_hbm, idx_v, rows_v, gsem0, gsem1):
    wid = lax.axis_index("s") * _NC + lax.axis_index("c")
    base = wid * _B_PER_W
    pltpu.sync_copy(idx_hbm.at[pl.ds(base, _B_PER_W)], idx_v)

    pltpu.async_copy(
        table_hbm.at[idx_v.at[pl.ds(0, _CHUNK)]], rows_v.at[0], gsem0
    )

    def body(p, _):
        c0 = 2 * p
        pltpu.async_copy(
            table_hbm.at[idx_v.at[pl.ds((c0 + 1) * _CHUNK, _CHUNK)]],
            rows_v.at[1],
            gsem1,
        )
        pltpu.make_async_copy(
            table_hbm.at[idx_v.at[pl.ds(0, _CHUNK)]], rows_v.at[0], gsem0
        ).wait()
        pltpu.sync_copy(
            rows_v.at[0], out_hbm.at[pl.ds(base + c0 * _CHUNK, _CHUNK)]
        )
        nxt = lax.min(c0 + 2, _NCHUNK - 2)
        pltpu.async_copy(
            table_hbm.at[idx_v.at[pl.ds(nxt * _CHUNK, _CHUNK)]],
            rows_v.at[0],
            gsem0,
        )
        pltpu.make_async_copy(
            table_hbm.at[idx_v.at[pl.ds(0, _CHUNK)]], rows_v.at[1], gsem1
        ).wait()
        pltpu.sync_copy(
            rows_v.at[1], out_hbm.at[pl.ds(base + (c0 + 1) * _CHUNK, _CHUNK)]
        )
        return _

    lax.fori_loop(0, _NCHUNK // 2, body, 0)
    pltpu.make_async_copy(
        table_hbm.at[idx_v.at[pl.ds(0, _CHUNK)]], rows_v.at[0], gsem0
    ).wait()


def kernel(input, weight):
    idxt = jnp.transpose(input.astype(jnp.int32))
    flat = _detile_idx(idxt)
    wt = jnp.transpose(weight)
    wtail = lax.slice(wt, (0, _VOCAB - 128), (_EMB, _VOCAB))
    lin = _linearize_table(wt, wtail)
    table = jnp.reshape(lin, (_VOCAB, _EMB))
    out = _emb_lookup(flat, table)
    out3 = jnp.reshape(out, (_HIST, _BATCH, _EMB))
    return jnp.transpose(out3, (1, 0, 2))


# speedup vs baseline: 4.3845x; 1.0012x over previous
import functools

import jax
import jax.numpy as jnp
from jax import lax
from jax.experimental import pallas as pl
from jax.experimental.pallas import tpu as pltpu
from jax.experimental.pallas import tpu_sc as plsc

_EMB = 64
_BATCH = 4096
_HIST = 50
_NTOT = _BATCH * _HIST
_VOCAB = 1000000

_info = plsc.get_sparse_core_info()
_NC, _NS = _info.num_cores, _info.num_subcores
_NW = _NC * _NS
_B_PER_W = _NTOT // _NW
_CHUNK = 800
_NCHUNK = _B_PER_W // _CHUNK

_NBLK_FULL = _VOCAB // 128
_BPW = 244
_BLK_F32 = 128 * _EMB

_mesh = plsc.VectorSubcoreMesh(core_axis_name="c", subcore_axis_name="s")


@functools.partial(
    pl.kernel,
    mesh=_mesh,
    out_type=jax.ShapeDtypeStruct((_NTOT,), jnp.int32),
    scratch_types=[
        pltpu.VMEM((8, 128), jnp.int32),
    ],
)
def _detile_idx(idxt_hbm, out_hbm, tile_v):
    wid = lax.axis_index("s") * _NC + lax.axis_index("c")
    col = wid * 128
    for a in range(7):
        rows = 8 if a < 6 else 2
        pltpu.sync_copy(
            idxt_hbm.at[pl.ds(a * 8, rows), pl.ds(col, 128)],
            tile_v.at[pl.ds(0, rows)],
        )
        for s in range(rows):
            pltpu.sync_copy(
                tile_v.at[s],
                out_hbm.at[pl.ds((a * 8 + s) * _BATCH + col, 128)],
            )


@functools.partial(
    pl.kernel,
    mesh=_mesh,
    out_type=jax.ShapeDtypeStruct((_VOCAB * _EMB,), jnp.float32),
    scratch_types=[
        pltpu.VMEM((2, _EMB, 129), jnp.float32),
        pltpu.VMEM((2, _BLK_F32), jnp.float32),
        pltpu.SemaphoreType.DMA,
        pltpu.SemaphoreType.DMA,
        pltpu.SemaphoreType.DMA,
        pltpu.SemaphoreType.DMA,
    ],
    compiler_params=pltpu.CompilerParams(needs_layout_passes=False),
)
def _linearize_table(wt_hbm, wtail_hbm, out_hbm, bin_v, bout_v, isem0, isem1,
                     osem0, osem1):
    wid = lax.axis_index("s") * _NC + lax.axis_index("c")
    blk0 = wid * _BPW
    iota16 = lax.iota(jnp.int32, 16)
    isems = (isem0, isem1)
    osems = (osem0, osem1)

    def start_in(j, b, tok=0):
        pltpu.async_copy(
            wt_hbm.at[:, pl.ds(pl.multiple_of(j * 128 + tok * 128, 128), 128)],
            bin_v.at[b, :, pl.ds(0, 128)],
            isems[b],
        )

    def wait_in(b):
        pltpu.make_async_copy(
            wt_hbm.at[:, pl.ds(0, 128)],
            bin_v.at[b, :, pl.ds(0, 128)],
            isems[b],
        ).wait()

    def start_out(j, b, tok=0):
        pltpu.async_copy(
            bout_v.at[b],
            out_hbm.at[
                pl.ds(pl.multiple_of(j * _BLK_F32 + tok * 8, 8), _BLK_F32)
            ],
            osems[b],
        )

    def wait_out(b):
        pltpu.make_async_copy(
            bout_v.at[b], out_hbm.at[pl.ds(0, _BLK_F32)], osems[b]
        ).wait()

    row_ids = [iota16 + (16 * s) for s in range(4)]

    def transpose(b):
        @functools.partial(plsc.parallel_loop, 0, 128, unroll=8)
        def _(col):
            cvec = jnp.full((16,), col, jnp.int32)
            for s in range(4):
                vals = plsc.load_gather(bin_v.at[b], [row_ids[s], cvec])
                bout_v[b, pl.ds(col * 64 + 16 * s, 16)] = vals

        chk = bout_v[b, pl.ds(0, 16)]
        return lax.bitcast_convert_type(jnp.max(chk), jnp.int32) & 0

    start_in(blk0, 0)
    start_in(blk0 + 1, 1)

    def iter_body(i, b, first):
        wait_in(b)
        if not first:
            wait_out(b)
        tok = transpose(b)
        start_out(blk0 + i, b, tok)

        @pl.when(i + 2 < _BPW)
        def _():
            start_in(blk0 + i + 2, b, tok)

    iter_body(0, 0, True)
    iter_body(1, 1, True)

    def body(k, _):
        iter_body(2 * k, 0, False)
        iter_body(2 * k + 1, 1, False)
        return _

    lax.fori_loop(1, _BPW // 2, body, 0)
    wait_out(0)
    wait_out(1)

    for w in range(4):

        @pl.when(wid == w)
        def _():
            jt = _NBLK_FULL - 4 + w
            start_in(jt, 0)
            wait_in(0)
            tok = transpose(0)
            start_out(jt, 0, tok)
            wait_out(0)

    @pl.when(wid == 4)
    def _():
        pltpu.sync_copy(wtail_hbm, bin_v.at[0, :, pl.ds(0, 128)])

        def tq(q, col):
            for p in range(2):
                cvec = col + p
                for s in range(4):
                    vals = plsc.load_gather(
                        bin_v.at[0], [iota16 + (16 * s), cvec]
                    )
                    bout_v[0, pl.ds(q * 128 + p * 64 + 16 * s, 16)] = vals
            return col + 2

        lax.fori_loop(0, 32, tq, jnp.full((16,), 64, jnp.int32))
        pltpu.sync_copy(
            bout_v.at[0, pl.ds(0, 32 * 128)],
            out_hbm.at[pl.ds(_VOCAB * _EMB - 32 * 128, 32 * 128)],
        )


@functools.partial(
    pl.kernel,
    mesh=_mesh,
    out_type=jax.ShapeDtypeStruct((_NTOT, _EMB), jnp.float32),
    scratch_types=[
        pltpu.VMEM((_B_PER_W,), jnp.int32),
        pltpu.VMEM((2, _CHUNK, _EMB), jnp.float32),
        pltpu.SemaphoreType.DMA,
        pltpu.SemaphoreType.DMA,
    ],
    compiler_params=pltpu.CompilerParams(use_tc_tiling_on_sc=False),
)
def _emb_lookup(idx_hbm, table_hbm, out_hbm, idx_v, rows_v, gsem0, gsem1):
    wid = lax.axis_index("s") * _NC + lax.axis_index("c")
    base = wid * _B_PER_W
    pltpu.sync_copy(idx_hbm.at[pl.ds(base, _B_PER_W)], idx_v)

    pltpu.async_copy(
        table_hbm.at[idx_v.at[pl.ds(0, _CHUNK)]], rows_v.at[0], gsem0
    )

    def body(p, _):
        c0 = 2 * p
        pltpu.async_copy(
            table_hbm.at[idx_v.at[pl.ds((c0 + 1) * _CHUNK, _CHUNK)]],
            rows_v.at[1],
            gsem1,
        )
        pltpu.make_async_copy(
            table_hbm.at[idx_v.at[pl.ds(0, _CHUNK)]], rows_v.at[0], gsem0
        ).wait()
        pltpu.sync_copy(
            rows_v.at[0], out_hbm.at[pl.ds(base + c0 * _CHUNK, _CHUNK)]
        )
        nxt = lax.min(c0 + 2, _NCHUNK - 2)
        pltpu.async_copy(
            table_hbm.at[idx_v.at[pl.ds(nxt * _CHUNK, _CHUNK)]],
            rows_v.at[0],
            gsem0,
        )
        pltpu.make_async_copy(
            table_hbm.at[idx_v.at[pl.ds(0, _CHUNK)]], rows_v.at[1], gsem1
        ).wait()
        pltpu.sync_copy(
            rows_v.at[1], out_hbm.at[pl.ds(base + (c0 + 1) * _CHUNK, _CHUNK)]
        )
        return _

    lax.fori_loop(0, _NCHUNK // 2, body, 0)
    pltpu.make_async_copy(
        table_hbm.at[idx_v.at[pl.ds(0, _CHUNK)]], rows_v.at[0], gsem0
    ).wait()


def kernel(input, weight):
    idxt = jnp.transpose(input.astype(jnp.int32))
    flat = _detile_idx(idxt)
    wt = jnp.transpose(weight)
    wtail = lax.slice(wt, (0, _VOCAB - 128), (_EMB, _VOCAB))
    lin = _linearize_table(wt, wtail)
    table = jnp.reshape(lin, (_VOCAB, _EMB))
    out = _emb_lookup(flat, table)
    out3 = jnp.reshape(out, (_HIST, _BATCH, _EMB))
    return jnp.transpose(out3, (1, 0, 2))
